# baseline (speedup 1.0000x reference)
import jax
import jax.numpy as jnp
from jax.experimental import pallas as pl
from jax.experimental.pallas import tpu as pltpu

_B, _C, _K, _H, _W = 64, 512, 64, 64, 64
_HW = _H * _W
_R = _HW // _C
_EPS = 1e-12


def _netvlad_kernel(xm_ref, w_ref, b_ref, cent_ref, out_ref, xn_scr):
    x4 = xm_ref[...].reshape(_C, _H, _W)
    xb4 = x4.astype(jnp.bfloat16)
    xb2 = xb4.reshape(_C, _HW)
    xs2 = xb2.astype(jnp.float32)
    ssq = jnp.sum(xs2 * xs2, axis=0, keepdims=True)
    rnorm = 1.0 / jnp.maximum(jnp.sqrt(ssq), _EPS)

    u = jax.lax.dot_general(
        w_ref[...].astype(jnp.bfloat16), xb2, (((1,), (0,)), ((), ())),
        preferred_element_type=jnp.float32)
    logits = u * rnorm + b_ref[...]
    m = jnp.max(logits, axis=0, keepdims=True)
    e = jnp.exp(logits - m)
    a = e / jnp.sum(e, axis=0, keepdims=True)

    xn_scr[...] = xb2 * rnorm.astype(jnp.bfloat16)
    xfn = xn_scr[...].reshape(_C, _R, _C).reshape(_HW, _C)

    vlad = jax.lax.dot_general(
        a.astype(jnp.bfloat16), xfn, (((1,), (0,)), ((), ())),
        preferred_element_type=jnp.float32)
    vlad = vlad - jnp.sum(a, axis=1, keepdims=True) * cent_ref[...]
    n1 = jnp.sqrt(jnp.sum(vlad * vlad, axis=1, keepdims=True))
    vlad = vlad / jnp.maximum(n1, _EPS)
    n2 = jnp.sqrt(jnp.sum(vlad * vlad))
    out_ref[0] = vlad / jnp.maximum(n2, _EPS)


def kernel(x, conv_w, conv_b, centroids):
    xm = x.reshape(_B * _C * _H, _W)
    out = pl.pallas_call(
        _netvlad_kernel,
        grid=(_B,),
        in_specs=[
            pl.BlockSpec((_C * _H, _W), lambda i: (i, 0)),
            pl.BlockSpec((_K, _C), lambda i: (0, 0)),
            pl.BlockSpec((_K, 1), lambda i: (0, 0)),
            pl.BlockSpec((_K, _C), lambda i: (0, 0)),
        ],
        out_specs=pl.BlockSpec((1, _K, _C), lambda i: (i, 0, 0)),
        out_shape=jax.ShapeDtypeStruct((_B, _K, _C), jnp.float32),
        scratch_shapes=[pltpu.VMEM((_C, _HW), jnp.bfloat16)],
        compiler_params=pltpu.CompilerParams(
            dimension_semantics=("parallel",),
            vmem_limit_bytes=56 * 1024 * 1024,
        ),
        name="netvlad_fused",
    )(xm, conv_w, conv_b.reshape(_K, 1), centroids)
    return out.reshape(_B, _K * _C)


# speedup vs baseline: 1.0318x; 1.0318x over previous
import jax
import jax.numpy as jnp
from jax.experimental import pallas as pl
from jax.experimental.pallas import tpu as pltpu

_B, _C, _K, _H, _W = 64, 512, 64, 64, 64
_HW = _H * _W
_R = _HW // _C
_EPS = 1e-12


def _netvlad_kernel(xm_ref, w_ref, b_ref, cent_ref, out_ref, xn_scr):
    x4 = xm_ref[...].reshape(_C, _H, _W)
    xb4 = x4.astype(jnp.bfloat16)
    xb2 = xb4.reshape(_C, _HW)
    xs2 = xb2.astype(jnp.float32)
    ssq = jnp.sum(xs2 * xs2, axis=0, keepdims=True)
    rnorm = 1.0 / jnp.maximum(jnp.sqrt(ssq), _EPS)

    xn_scr[...] = xb2 * rnorm.astype(jnp.bfloat16)

    logits = jax.lax.dot_general(
        w_ref[...].astype(jnp.bfloat16), xn_scr[...], (((1,), (0,)), ((), ())),
        preferred_element_type=jnp.float32) + b_ref[...]
    m = jnp.max(logits, axis=0, keepdims=True)
    e = jnp.exp(logits - m)
    a = e / jnp.sum(e, axis=0, keepdims=True)

    xfn = xn_scr[...].reshape(_C, _R, _C).reshape(_HW, _C)

    vlad = jax.lax.dot_general(
        a.astype(jnp.bfloat16), xfn, (((1,), (0,)), ((), ())),
        preferred_element_type=jnp.float32)
    vlad = vlad - jnp.sum(a, axis=1, keepdims=True) * cent_ref[...]
    n1 = jnp.sqrt(jnp.sum(vlad * vlad, axis=1, keepdims=True))
    vlad = vlad / jnp.maximum(n1, _EPS)
    n2 = jnp.sqrt(jnp.sum(vlad * vlad))
    out_ref[0] = vlad / jnp.maximum(n2, _EPS)


def kernel(x, conv_w, conv_b, centroids):
    xm = x.reshape(_B * _C * _H, _W)
    out = pl.pallas_call(
        _netvlad_kernel,
        grid=(_B,),
        in_specs=[
            pl.BlockSpec((_C * _H, _W), lambda i: (i, 0)),
            pl.BlockSpec((_K, _C), lambda i: (0, 0)),
            pl.BlockSpec((_K, 1), lambda i: (0, 0)),
            pl.BlockSpec((_K, _C), lambda i: (0, 0)),
        ],
        out_specs=pl.BlockSpec((1, _K, _C), lambda i: (i, 0, 0)),
        out_shape=jax.ShapeDtypeStruct((_B, _K, _C), jnp.float32),
        scratch_shapes=[pltpu.VMEM((_C, _HW), jnp.bfloat16)],
        compiler_params=pltpu.CompilerParams(
            dimension_semantics=("parallel",),
            vmem_limit_bytes=56 * 1024 * 1024,
        ),
        name="netvlad_fused",
    )(xm, conv_w, conv_b.reshape(_K, 1), centroids)
    return out.reshape(_B, _K * _C)
